# baseline (speedup 1.0000x reference)
import jax
import jax.numpy as jnp
from jax.experimental import pallas as pl
from jax.experimental.pallas import tpu as pltpu

_NUM_CLASSES = 1000
_TILE_N = 128


def _linear_kernel(x_ref, wt_ref, b_ref, o_ref, wbf_ref):
    i = pl.program_id(0)
    j = pl.program_id(1)

    @pl.when((i == 0) & (j == 0))
    def _():
        wbf_ref[...] = wt_ref[...].astype(jnp.bfloat16)

    tn = o_ref.shape[1]
    x = x_ref[...].astype(jnp.bfloat16)
    w = wbf_ref[:, pl.ds(j * tn, tn)]
    acc = jnp.dot(x, w, preferred_element_type=jnp.float32)
    o_ref[...] = acc + b_ref[:, pl.ds(j * tn, tn)]


def kernel(x, wt_p, b_p):
    M, K = x.shape
    K_pad, N_pad = wt_p.shape
    n = min(_NUM_CLASSES, N_pad)

    tile_m = next(t for t in (1024, 512, 256, 128, 64, 8, 1) if M % t == 0)
    m_steps = M // tile_m
    n_steps = (n + _TILE_N - 1) // _TILE_N

    cost = pl.CostEstimate(
        flops=2 * M * K_pad * N_pad,
        transcendentals=0,
        bytes_accessed=M * K * 4 + K_pad * N_pad * 4 + N_pad * 4 + M * n * 4,
    )

    return pl.pallas_call(
        _linear_kernel,
        out_shape=jax.ShapeDtypeStruct((M, n), x.dtype),
        grid=(m_steps, n_steps),
        in_specs=[
            pl.BlockSpec((tile_m, K), lambda i, j: (i, 0)),
            pl.BlockSpec((K_pad, N_pad), lambda i, j: (0, 0)),
            pl.BlockSpec((1, N_pad), lambda i, j: (0, 0)),
        ],
        out_specs=pl.BlockSpec((tile_m, _TILE_N), lambda i, j: (i, j)),
        scratch_shapes=[pltpu.VMEM((K_pad, N_pad), jnp.bfloat16)],
        compiler_params=pltpu.CompilerParams(
            dimension_semantics=("arbitrary", "arbitrary"),
        ),
        cost_estimate=cost,
    )(x, wt_p, b_p)


# speedup vs baseline: 1.8606x; 1.8606x over previous
import jax
import jax.numpy as jnp
from jax.experimental import pallas as pl
from jax.experimental.pallas import tpu as pltpu

_NUM_CLASSES = 1000


def _out_copies(o_ref, acc_ref, tail_ref, sem_ref, t, tile_m, n_al, n):
    s = jax.lax.rem(t, 2)
    rows = pl.ds(t * tile_m, tile_m)
    copies = [
        pltpu.make_async_copy(
            acc_ref.at[s, :, pl.ds(0, n_al)],
            o_ref.at[rows, pl.ds(0, n_al)],
            sem_ref.at[s, 0],
        )
    ]
    if n > n_al:
        copies.append(
            pltpu.make_async_copy(
                tail_ref.at[s],
                o_ref.at[rows, pl.ds(n_al, n - n_al)],
                sem_ref.at[s, 1],
            )
        )
    return copies


def _linear_kernel(x_ref, wt_ref, b_ref, o_ref, wbf_ref, acc_ref, tail_ref,
                   sem_ref):
    i = pl.program_id(0)
    nsteps = pl.num_programs(0)
    slot = jax.lax.rem(i, 2)
    tile_m = x_ref.shape[0]
    n = o_ref.shape[1]
    n_al = (n // 128) * 128

    @pl.when(i == 0)
    def _():
        wbf_ref[...] = wt_ref[...].astype(jnp.bfloat16)

    @pl.when(i >= 2)
    def _():
        for c in _out_copies(o_ref, acc_ref, tail_ref, sem_ref, i - 2,
                             tile_m, n_al, n):
            c.wait()

    x = x_ref[...].astype(jnp.bfloat16)
    acc = jnp.dot(x, wbf_ref[...], preferred_element_type=jnp.float32)
    acc = acc + b_ref[...]
    acc_ref[slot] = acc
    if n > n_al:
        tail_ref[slot] = acc[:, n_al:n]

    for c in _out_copies(o_ref, acc_ref, tail_ref, sem_ref, i,
                         tile_m, n_al, n):
        c.start()

    @pl.when(i == nsteps - 1)
    def _():
        @pl.when(nsteps >= 2)
        def _():
            for c in _out_copies(o_ref, acc_ref, tail_ref, sem_ref, i - 1,
                                 tile_m, n_al, n):
                c.wait()

        for c in _out_copies(o_ref, acc_ref, tail_ref, sem_ref, i,
                             tile_m, n_al, n):
            c.wait()


def kernel(x, wt_p, b_p):
    M, K = x.shape
    K_pad, N_pad = wt_p.shape
    n = min(_NUM_CLASSES, N_pad)
    n_al = (n // 128) * 128
    n_tail = max(n - n_al, 8)

    tile_m = next(t for t in (1024, 512, 256, 128, 64, 8, 1) if M % t == 0)
    m_steps = M // tile_m

    cost = pl.CostEstimate(
        flops=2 * M * K_pad * N_pad,
        transcendentals=0,
        bytes_accessed=M * K * 4 + K_pad * N_pad * 4 + N_pad * 4 + M * n * 4,
    )

    return pl.pallas_call(
        _linear_kernel,
        out_shape=jax.ShapeDtypeStruct((M, n), x.dtype),
        grid=(m_steps,),
        in_specs=[
            pl.BlockSpec((tile_m, K), lambda i: (i, 0)),
            pl.BlockSpec((K_pad, N_pad), lambda i: (0, 0)),
            pl.BlockSpec((1, N_pad), lambda i: (0, 0)),
        ],
        out_specs=pl.BlockSpec(memory_space=pl.ANY),
        scratch_shapes=[
            pltpu.VMEM((K_pad, N_pad), jnp.bfloat16),
            pltpu.VMEM((2, tile_m, N_pad), jnp.float32),
            pltpu.VMEM((2, tile_m, n_tail), jnp.float32),
            pltpu.SemaphoreType.DMA((2, 2)),
        ],
        compiler_params=pltpu.CompilerParams(
            dimension_semantics=("arbitrary",),
        ),
        cost_estimate=cost,
    )(x, wt_p, b_p)
